# baseline (speedup 1.0000x reference)
import functools

import jax
import jax.numpy as jnp
from jax import lax
from jax.experimental import pallas as pl
from jax.experimental.pallas import tpu as pltpu
from jax.experimental.pallas import tpu_sc as plsc

N = 10000
E = 320000
D = 128
Q = D // 4
NC, NS = 2, 16
NW = NC * NS
K = 128

NROW = 10240

CA = 158
AGG_PAD_SRC = N
ACC_ROWS = 10112
ACC_TILE = ACC_ROWS // NS
AGG_PAD_DST = N

CD = 79
DEG_ROWS = 10112
DEG_TILE = DEG_ROWS // NS
DEG_PAD_DST = N

_mesh = plsc.VectorSubcoreMesh(
    core_axis_name="c", subcore_axis_name="s", num_cores=NC, num_subcores=NS)
_sc_params = pltpu.CompilerParams(use_tc_tiling_on_sc=False)


@functools.partial(
    pl.kernel,
    out_type=jax.ShapeDtypeStruct((NC * DEG_ROWS, Q), jnp.float32),
    mesh=_mesh,
    scratch_types=[
        pltpu.VMEM((CD, K), jnp.int32),
        pltpu.VMEM((K, Q), jnp.float32),
        pltpu.VMEM_SHARED((DEG_ROWS, Q), jnp.float32),
    ],
    compiler_params=_sc_params,
)
def _deg_kernel(dst_hbm, ones_hbm, zeros_hbm, degp_hbm, didx_v, ones_v, deg_sh):
    c = lax.axis_index("c")
    s = lax.axis_index("s")
    w = c * NS + s
    pltpu.sync_copy(zeros_hbm, deg_sh.at[pl.ds(s * DEG_TILE, DEG_TILE)])
    pltpu.sync_copy(ones_hbm, ones_v)
    pltpu.sync_copy(dst_hbm.at[w], didx_v)
    plsc.subcore_barrier()

    @pl.loop(0, CD)
    def _(j):
        pltpu.sync_copy(ones_v, deg_sh.at[didx_v.at[j]], add=True)

    plsc.subcore_barrier()
    pltpu.sync_copy(deg_sh.at[pl.ds(s * DEG_TILE, DEG_TILE)],
                    degp_hbm.at[pl.ds(c * DEG_ROWS + s * DEG_TILE, DEG_TILE)])


@functools.partial(
    pl.kernel,
    out_type=jax.ShapeDtypeStruct((4 * ACC_ROWS, Q), jnp.float32),
    mesh=_mesh,
    scratch_types=[
        pltpu.VMEM((CA + 4, K), jnp.int32),
        pltpu.VMEM((CA + 4, K), jnp.int32),
    ] + [pltpu.VMEM((K, Q), jnp.float32)] * 6 + [
        pltpu.VMEM_SHARED((ACC_ROWS, Q), jnp.float32),
    ] + [pltpu.SemaphoreType.DMA] * 6,
    compiler_params=_sc_params,
)
def _agg_kernel(hp_hbm, src_hbm, dst_hbm, zeros_hbm, outp_hbm,
                sidx_v, didx_v, b0, b1, b2, b3, b4, b5,
                acc_sh, s0, s1, s2, s3, s4, s5):
    c = lax.axis_index("c")
    s = lax.axis_index("s")
    pltpu.sync_copy(dst_hbm.at[s], didx_v)
    bufs = (b0, b1, b2, b3, b4, b5)
    sems = (s0, s1, s2, s3, s4, s5)

    def _gather(j, b):
        return pltpu.async_copy(hp_hbm.at[sidx_v.at[j]], bufs[b], sems[b])

    def _gather_wait(j, b):
        pltpu.make_async_copy(hp_hbm.at[sidx_v.at[j]], bufs[b], sems[b]).wait()

    def _scatter(j, b):
        return pltpu.async_copy(bufs[b], acc_sh.at[didx_v.at[j]], sems[b],
                                add=True)

    def _scatter_wait(j, b):
        pltpu.make_async_copy(bufs[b], acc_sh.at[didx_v.at[j]], sems[b]).wait()

    for p in range(2):
        q = c * 2 + p
        pltpu.sync_copy(zeros_hbm, acc_sh.at[pl.ds(s * ACC_TILE, ACC_TILE)])
        pltpu.sync_copy(src_hbm.at[q * NS + s], sidx_v)
        plsc.subcore_barrier()

        for j in range(4):
            _gather(j, j)
        for j in range(2):
            _gather_wait(j, j)
            _gather(j + 4, j + 4)
            _scatter(j, j)

        @pl.loop(2, CA, step=6)
        def _(j):
            for k in range(6):
                jj = j + k
                b = (2 + k) % 6
                _gather_wait(jj, b)
                _scatter_wait(jj - 2, (b + 4) % 6)
                _gather(jj + 4, (b + 4) % 6)
                _scatter(jj, b)

        for j in range(CA - 2, CA):
            _scatter_wait(j, j % 6)
        for j in range(CA, CA + 4):
            _gather_wait(j, j % 6)
        plsc.subcore_barrier()
        pltpu.sync_copy(acc_sh.at[pl.ds(s * ACC_TILE, ACC_TILE)],
                        outp_hbm.at[pl.ds(q * ACC_ROWS + s * ACC_TILE, ACC_TILE)])


_R = 2048
_GRID = NROW // _R

_x_spec = pl.BlockSpec((_R, D), lambda i: (i, 0))
_hp_spec = pl.BlockSpec((4, _R, Q), lambda i: (0, i, 0))


def _q_spec(q):
    return pl.BlockSpec((1, _R, Q), lambda i, _q=q: (_q, i, 0))


_q_specs = [_q_spec(q) for q in range(4)]
_deg0_spec = pl.BlockSpec((1, _R, Q), lambda i: (0, i, 0))
_deg1_spec = pl.BlockSpec((1, _R, Q), lambda i: (1, i, 0))
_w_spec = pl.BlockSpec((D, D), lambda i: (0, 0))
_b_spec = pl.BlockSpec((1, D), lambda i: (0, 0))


def _valid_rows(nrows):
    rid = lax.broadcasted_iota(jnp.int32, (_R, 1), 0) + pl.program_id(0) * _R
    return rid < nrows


def _ds_of(d0_ref, d1_ref):
    return lax.rsqrt(d0_ref[0][:, :1] + d1_ref[0][:, :1] + 1.0)


def _split_out(out_ref, hp):
    for q in range(4):
        out_ref[q] = hp[:, q * Q:(q + 1) * Q]


def _tc1_body(x_ref, d0_ref, d1_ref, w_ref, out_ref):
    ds = _ds_of(d0_ref, d1_ref)
    h = lax.dot_general(x_ref[...], w_ref[...], (((1,), (1,)), ((), ())),
                        preferred_element_type=jnp.float32)
    _split_out(out_ref, jnp.where(_valid_rows(N), ds * h, 0.0))


_tc1 = pl.pallas_call(
    _tc1_body,
    grid=(_GRID,),
    in_specs=[_x_spec, _deg0_spec, _deg1_spec, _w_spec],
    out_specs=_hp_spec,
    out_shape=jax.ShapeDtypeStruct((4, NROW, Q), jnp.float32),
)


def _tc2_body(a0, a1, a2, a3, h0, h1, h2, h3, d0_ref, d1_ref, b_ref, w_ref,
              out_ref):
    ds = _ds_of(d0_ref, d1_ref)
    z = jnp.concatenate(
        [a0[0] + h0[0], a1[0] + h1[0], a2[0] + h2[0], a3[0] + h3[0]], axis=1)
    h = jnp.tanh(ds * z + b_ref[...])
    hp = ds * lax.dot_general(h, w_ref[...], (((1,), (1,)), ((), ())),
                              preferred_element_type=jnp.float32)
    _split_out(out_ref, jnp.where(_valid_rows(N), hp, 0.0))


_tc2 = pl.pallas_call(
    _tc2_body,
    grid=(_GRID,),
    in_specs=_q_specs + _q_specs + [_deg0_spec, _deg1_spec, _b_spec, _w_spec],
    out_specs=_hp_spec,
    out_shape=jax.ShapeDtypeStruct((4, NROW, Q), jnp.float32),
)


def _tc3_body(a0, a1, a2, a3, h0, h1, h2, h3, d0_ref, d1_ref, b_ref, w_ref,
              br_ref, out_ref):
    ds = _ds_of(d0_ref, d1_ref)
    z = jnp.concatenate(
        [a0[0] + h0[0], a1[0] + h1[0], a2[0] + h2[0], a3[0] + h3[0]], axis=1)
    h = jnp.tanh(ds * z + b_ref[...])
    y = lax.dot_general(h, w_ref[...], (((1,), (1,)), ((), ())),
                        preferred_element_type=jnp.float32) + br_ref[...]
    out_ref[...] = jax.nn.sigmoid(y) * 0.8 + 0.1


_tc3 = pl.pallas_call(
    _tc3_body,
    grid=(_GRID,),
    in_specs=_q_specs + _q_specs + [_deg0_spec, _deg1_spec, _b_spec, _w_spec,
                                    _b_spec],
    out_specs=pl.BlockSpec((_R, D), lambda i: (i, 0)),
    out_shape=jax.ShapeDtypeStruct((N, D), jnp.float32),
)


def kernel(x, edge_index, W1, b1, W2, b2, Wr, br):
    src = edge_index[0].astype(jnp.int32)
    dst = edge_index[1].astype(jnp.int32)

    padd = NW * CD * K - E
    dstd = jnp.concatenate([dst, jnp.full((padd,), DEG_PAD_DST, jnp.int32)])
    dstd = dstd.reshape(NW, CD, K)

    pada = NS * CA * K - E
    srca = jnp.concatenate([src, jnp.full((pada,), AGG_PAD_SRC, jnp.int32)])
    srca = srca.reshape(NS, CA, K)
    dsta = jnp.concatenate([dst, jnp.full((pada,), AGG_PAD_DST, jnp.int32)])
    dsta = dsta.reshape(NS, CA, K)
    extra = jnp.full((NS, 4, K), AGG_PAD_SRC, jnp.int32)
    srca = jnp.concatenate([srca, extra], axis=1)
    dsta = jnp.concatenate([dsta, extra], axis=1)
    srca = jnp.concatenate([srca + q * NROW for q in range(4)], axis=0)

    xp = jnp.concatenate([x, jnp.zeros((NROW - N, D), jnp.float32)])

    onesQ = jnp.ones((K, Q), jnp.float32)
    zerosQ = jnp.zeros((DEG_TILE, Q), jnp.float32)
    b1r = b1.reshape(1, D)
    b2r = b2.reshape(1, D)
    brr = br.reshape(1, D)

    degp = _deg_kernel(dstd, onesQ, zerosQ).reshape(NC, DEG_ROWS, Q)
    h1p = _tc1(xp, degp, degp, W1)
    acc1 = _agg_kernel(h1p.reshape(4 * NROW, Q), srca, dsta,
                       zerosQ).reshape(4, ACC_ROWS, Q)
    h2p = _tc2(acc1, acc1, acc1, acc1, h1p, h1p, h1p, h1p,
               degp, degp, b1r, W2)
    acc2 = _agg_kernel(h2p.reshape(4 * NROW, Q), srca, dsta,
                       zerosQ).reshape(4, ACC_ROWS, Q)
    return _tc3(acc2, acc2, acc2, acc2, h2p, h2p, h2p, h2p,
                degp, degp, b2r, Wr, brr)


# speedup vs baseline: 1.1074x; 1.1074x over previous
import functools

import jax
import jax.numpy as jnp
from jax import lax
from jax.experimental import pallas as pl
from jax.experimental.pallas import tpu as pltpu
from jax.experimental.pallas import tpu_sc as plsc

N = 10000
E = 320000
D = 128
Q = D // 4
NC, NS = 2, 16
NW = NC * NS
K = 128

NROW = 10240

CA = 158
AGG_PAD_SRC = N
ACC_ROWS = 10112
ACC_TILE = ACC_ROWS // NS
AGG_PAD_DST = N

CD = 79
DEG_ROWS = 10112
DEG_TILE = DEG_ROWS // NS
DEG_PAD_DST = N

_mesh = plsc.VectorSubcoreMesh(
    core_axis_name="c", subcore_axis_name="s", num_cores=NC, num_subcores=NS)
_sc_params = pltpu.CompilerParams(use_tc_tiling_on_sc=False)


@functools.partial(
    pl.kernel,
    out_type=jax.ShapeDtypeStruct((NC * DEG_ROWS, Q), jnp.float32),
    mesh=_mesh,
    scratch_types=[
        pltpu.VMEM((CD, K), jnp.int32),
        pltpu.VMEM((K, Q), jnp.float32),
        pltpu.VMEM_SHARED((DEG_ROWS, Q), jnp.float32),
    ],
    compiler_params=_sc_params,
)
def _deg_kernel(dst_hbm, ones_hbm, zeros_hbm, degp_hbm, didx_v, ones_v, deg_sh):
    c = lax.axis_index("c")
    s = lax.axis_index("s")
    w = c * NS + s
    pltpu.sync_copy(zeros_hbm, deg_sh.at[pl.ds(s * DEG_TILE, DEG_TILE)])
    pltpu.sync_copy(ones_hbm, ones_v)
    pltpu.sync_copy(dst_hbm.at[w], didx_v)
    plsc.subcore_barrier()

    @pl.loop(0, CD)
    def _(j):
        pltpu.sync_copy(ones_v, deg_sh.at[didx_v.at[j]], add=True)

    plsc.subcore_barrier()
    pltpu.sync_copy(deg_sh.at[pl.ds(s * DEG_TILE, DEG_TILE)],
                    degp_hbm.at[pl.ds(c * DEG_ROWS + s * DEG_TILE, DEG_TILE)])


@functools.partial(
    pl.kernel,
    out_type=jax.ShapeDtypeStruct((4 * ACC_ROWS, Q), jnp.float32),
    mesh=_mesh,
    scratch_types=[
        pltpu.VMEM((CA + 4, K), jnp.int32),
        pltpu.VMEM((CA + 4, K), jnp.int32),
    ] + [pltpu.VMEM((K, Q), jnp.float32)] * 4 + [
        pltpu.VMEM_SHARED((ACC_ROWS, Q), jnp.float32),
    ] + [pltpu.SemaphoreType.DMA] * 4,
    compiler_params=_sc_params,
)
def _agg_kernel(hp_hbm, src_hbm, dst_hbm, zeros_hbm, outp_hbm,
                sidx_v, didx_v, b0, b1, b2, b3,
                acc_sh, s0, s1, s2, s3):
    c = lax.axis_index("c")
    s = lax.axis_index("s")
    pltpu.sync_copy(dst_hbm.at[s], didx_v)
    bufs = (b0, b1, b2, b3)
    sems = (s0, s1, s2, s3)

    def _gather(j, b):
        return pltpu.async_copy(hp_hbm.at[sidx_v.at[j]], bufs[b], sems[b])

    def _gather_wait(j, b):
        pltpu.make_async_copy(hp_hbm.at[sidx_v.at[j]], bufs[b], sems[b]).wait()

    def _scatter(j, b):
        return pltpu.async_copy(bufs[b], acc_sh.at[didx_v.at[j]], sems[b],
                                add=True)

    def _scatter_wait(j, b):
        pltpu.make_async_copy(bufs[b], acc_sh.at[didx_v.at[j]], sems[b]).wait()

    for p in range(2):
        q = c * 2 + p
        pltpu.sync_copy(zeros_hbm, acc_sh.at[pl.ds(s * ACC_TILE, ACC_TILE)])
        pltpu.sync_copy(src_hbm.at[q * NS + s], sidx_v)
        plsc.subcore_barrier()

        for j in range(2):
            _gather(j, j)
        for j in range(2):
            _gather_wait(j, j)
            _gather(j + 2, j + 2)
            _scatter(j, j)

        @pl.loop(2, CA, step=4)
        def _(j):
            for k in range(4):
                jj = j + k
                b = (2 + k) % 4
                _gather_wait(jj, b)
                _scatter_wait(jj - 2, (b + 2) % 4)
                _gather(jj + 2, (b + 2) % 4)
                _scatter(jj, b)

        for j in range(CA - 2, CA):
            _scatter_wait(j, j % 4)
        for j in range(CA, CA + 2):
            _gather_wait(j, j % 4)
        plsc.subcore_barrier()
        pltpu.sync_copy(acc_sh.at[pl.ds(s * ACC_TILE, ACC_TILE)],
                        outp_hbm.at[pl.ds(q * ACC_ROWS + s * ACC_TILE, ACC_TILE)])


_R = 2048
_GRID = NROW // _R

_x_spec = pl.BlockSpec((_R, D), lambda i: (i, 0))
_hp_spec = pl.BlockSpec((4, _R, Q), lambda i: (0, i, 0))


def _q_spec(q):
    return pl.BlockSpec((1, _R, Q), lambda i, _q=q: (_q, i, 0))


_q_specs = [_q_spec(q) for q in range(4)]
_deg0_spec = pl.BlockSpec((1, _R, Q), lambda i: (0, i, 0))
_deg1_spec = pl.BlockSpec((1, _R, Q), lambda i: (1, i, 0))
_w_spec = pl.BlockSpec((D, D), lambda i: (0, 0))
_b_spec = pl.BlockSpec((1, D), lambda i: (0, 0))


def _valid_rows(nrows):
    rid = lax.broadcasted_iota(jnp.int32, (_R, 1), 0) + pl.program_id(0) * _R
    return rid < nrows


def _ds_of(d0_ref, d1_ref):
    return lax.rsqrt(d0_ref[0][:, :1] + d1_ref[0][:, :1] + 1.0)


def _split_out(out_ref, hp):
    for q in range(4):
        out_ref[q] = hp[:, q * Q:(q + 1) * Q]


def _tc1_body(x_ref, d0_ref, d1_ref, w_ref, out_ref):
    ds = _ds_of(d0_ref, d1_ref)
    h = lax.dot_general(x_ref[...], w_ref[...], (((1,), (1,)), ((), ())),
                        preferred_element_type=jnp.float32)
    _split_out(out_ref, jnp.where(_valid_rows(N), ds * h, 0.0))


_tc1 = pl.pallas_call(
    _tc1_body,
    grid=(_GRID,),
    in_specs=[_x_spec, _deg0_spec, _deg1_spec, _w_spec],
    out_specs=_hp_spec,
    out_shape=jax.ShapeDtypeStruct((4, NROW, Q), jnp.float32),
)


def _tc2_body(a0, a1, a2, a3, h0, h1, h2, h3, d0_ref, d1_ref, b_ref, w_ref,
              out_ref):
    ds = _ds_of(d0_ref, d1_ref)
    z = jnp.concatenate(
        [a0[0] + h0[0], a1[0] + h1[0], a2[0] + h2[0], a3[0] + h3[0]], axis=1)
    h = jnp.tanh(ds * z + b_ref[...])
    hp = ds * lax.dot_general(h, w_ref[...], (((1,), (1,)), ((), ())),
                              preferred_element_type=jnp.float32)
    _split_out(out_ref, jnp.where(_valid_rows(N), hp, 0.0))


_tc2 = pl.pallas_call(
    _tc2_body,
    grid=(_GRID,),
    in_specs=_q_specs + _q_specs + [_deg0_spec, _deg1_spec, _b_spec, _w_spec],
    out_specs=_hp_spec,
    out_shape=jax.ShapeDtypeStruct((4, NROW, Q), jnp.float32),
)


def _tc3_body(a0, a1, a2, a3, h0, h1, h2, h3, d0_ref, d1_ref, b_ref, w_ref,
              br_ref, out_ref):
    ds = _ds_of(d0_ref, d1_ref)
    z = jnp.concatenate(
        [a0[0] + h0[0], a1[0] + h1[0], a2[0] + h2[0], a3[0] + h3[0]], axis=1)
    h = jnp.tanh(ds * z + b_ref[...])
    y = lax.dot_general(h, w_ref[...], (((1,), (1,)), ((), ())),
                        preferred_element_type=jnp.float32) + br_ref[...]
    out_ref[...] = jax.nn.sigmoid(y) * 0.8 + 0.1


_tc3 = pl.pallas_call(
    _tc3_body,
    grid=(_GRID,),
    in_specs=_q_specs + _q_specs + [_deg0_spec, _deg1_spec, _b_spec, _w_spec,
                                    _b_spec],
    out_specs=pl.BlockSpec((_R, D), lambda i: (i, 0)),
    out_shape=jax.ShapeDtypeStruct((N, D), jnp.float32),
)


def kernel(x, edge_index, W1, b1, W2, b2, Wr, br):
    src = edge_index[0].astype(jnp.int32)
    dst = edge_index[1].astype(jnp.int32)

    padd = NW * CD * K - E
    dstd = jnp.concatenate([dst, jnp.full((padd,), DEG_PAD_DST, jnp.int32)])
    dstd = dstd.reshape(NW, CD, K)

    pada = NS * CA * K - E
    srca = jnp.concatenate([src, jnp.full((pada,), AGG_PAD_SRC, jnp.int32)])
    srca = srca.reshape(NS, CA, K)
    dsta = jnp.concatenate([dst, jnp.full((pada,), AGG_PAD_DST, jnp.int32)])
    dsta = dsta.reshape(NS, CA, K)
    extra = jnp.full((NS, 4, K), AGG_PAD_SRC, jnp.int32)
    srca = jnp.concatenate([srca, extra], axis=1)
    dsta = jnp.concatenate([dsta, extra], axis=1)
    srca = jnp.concatenate([srca + q * NROW for q in range(4)], axis=0)

    xp = jnp.concatenate([x, jnp.zeros((NROW - N, D), jnp.float32)])

    onesQ = jnp.ones((K, Q), jnp.float32)
    zerosQ = jnp.zeros((DEG_TILE, Q), jnp.float32)
    b1r = b1.reshape(1, D)
    b2r = b2.reshape(1, D)
    brr = br.reshape(1, D)

    degp = _deg_kernel(dstd, onesQ, zerosQ).reshape(NC, DEG_ROWS, Q)
    h1p = _tc1(xp, degp, degp, W1)
    acc1 = _agg_kernel(h1p.reshape(4 * NROW, Q), srca, dsta,
                       zerosQ).reshape(4, ACC_ROWS, Q)
    h2p = _tc2(acc1, acc1, acc1, acc1, h1p, h1p, h1p, h1p,
               degp, degp, b1r, W2)
    acc2 = _agg_kernel(h2p.reshape(4 * NROW, Q), srca, dsta,
                       zerosQ).reshape(4, ACC_ROWS, Q)
    return _tc3(acc2, acc2, acc2, acc2, h2p, h2p, h2p, h2p,
                degp, degp, b2r, Wr, brr)


# speedup vs baseline: 1.2651x; 1.1424x over previous
import functools

import jax
import jax.numpy as jnp
from jax import lax
from jax.experimental import pallas as pl
from jax.experimental.pallas import tpu as pltpu
from jax.experimental.pallas import tpu_sc as plsc

N = 10000
E = 320000
D = 128
H = D // 2
Q = D // 4
NC, NS = 2, 16
NW = NC * NS
K = 128

NROW = 10240

CA = 158
AGG_PAD_SRC = N
ACC_ROWS = 10112
ACC_TILE = ACC_ROWS // NS
AGG_PAD_DST = N

CD = 79
DEG_ROWS = 10112
DEG_TILE = DEG_ROWS // NS
DEG_PAD_DST = N

_mesh = plsc.VectorSubcoreMesh(
    core_axis_name="c", subcore_axis_name="s", num_cores=NC, num_subcores=NS)
_sc_params = pltpu.CompilerParams(use_tc_tiling_on_sc=False,
                                  internal_scratch_in_bytes=262144)


@functools.partial(
    pl.kernel,
    out_type=jax.ShapeDtypeStruct((NC * DEG_ROWS, Q), jnp.float32),
    mesh=_mesh,
    scratch_types=[
        pltpu.VMEM((CD, K), jnp.int32),
        pltpu.VMEM((K, Q), jnp.float32),
        pltpu.VMEM_SHARED((DEG_ROWS, Q), jnp.float32),
    ],
    compiler_params=_sc_params,
)
def _deg_kernel(dst_hbm, ones_hbm, zeros_hbm, degp_hbm, didx_v, ones_v, deg_sh):
    c = lax.axis_index("c")
    s = lax.axis_index("s")
    w = c * NS + s
    pltpu.sync_copy(zeros_hbm, deg_sh.at[pl.ds(s * DEG_TILE, DEG_TILE)])
    pltpu.sync_copy(ones_hbm, ones_v)
    pltpu.sync_copy(dst_hbm.at[w], didx_v)
    plsc.subcore_barrier()

    @pl.loop(0, CD)
    def _(j):
        pltpu.sync_copy(ones_v, deg_sh.at[didx_v.at[j]], add=True)

    plsc.subcore_barrier()
    pltpu.sync_copy(deg_sh.at[pl.ds(s * DEG_TILE, DEG_TILE)],
                    degp_hbm.at[pl.ds(c * DEG_ROWS + s * DEG_TILE, DEG_TILE)])


@functools.partial(
    pl.kernel,
    out_type=jax.ShapeDtypeStruct((NC * ACC_ROWS, H), jnp.float32),
    mesh=_mesh,
    scratch_types=[
        pltpu.VMEM((CA + 2, K), jnp.int32),
        pltpu.VMEM((CA + 2, K), jnp.int32),
    ] + [pltpu.VMEM((K, H), jnp.float32)] * 4 + [
        pltpu.VMEM_SHARED((ACC_ROWS, H), jnp.float32),
    ] + [pltpu.SemaphoreType.DMA] * 4,
    compiler_params=_sc_params,
)
def _agg_kernel(hp_hbm, src_hbm, dst_hbm, zeros_hbm, outp_hbm,
                sidx_v, didx_v, b0, b1, b2, b3,
                acc_sh, s0, s1, s2, s3):
    c = lax.axis_index("c")
    s = lax.axis_index("s")
    bufs = (b0, b1, b2, b3)
    sems = (s0, s1, s2, s3)

    def _gather(j, b):
        return pltpu.async_copy(hp_hbm.at[sidx_v.at[j]], bufs[b], sems[b])

    def _gather_wait(j, b):
        pltpu.make_async_copy(hp_hbm.at[sidx_v.at[j]], bufs[b], sems[b]).wait()

    def _scatter(j, b):
        return pltpu.async_copy(bufs[b], acc_sh.at[didx_v.at[j]], sems[b],
                                add=True)

    def _scatter_wait(j, b):
        pltpu.make_async_copy(bufs[b], acc_sh.at[didx_v.at[j]], sems[b]).wait()

    pltpu.sync_copy(dst_hbm.at[s], didx_v)
    pltpu.sync_copy(zeros_hbm, acc_sh.at[pl.ds(s * ACC_TILE, ACC_TILE)])
    pltpu.sync_copy(src_hbm.at[c * NS + s], sidx_v)
    plsc.subcore_barrier()

    for j in range(2):
        _gather(j, j)
    for j in range(2):
        _gather_wait(j, j)
        _gather(j + 2, j + 2)
        _scatter(j, j)

    @pl.loop(2, CA, step=4)
    def _(j):
        for k in range(4):
            jj = j + k
            b = (2 + k) % 4
            _gather_wait(jj, b)
            _scatter_wait(jj - 2, (b + 2) % 4)
            _gather(jj + 2, (b + 2) % 4)
            _scatter(jj, b)

    for j in range(CA - 2, CA):
        _scatter_wait(j, j % 4)
    for j in range(CA, CA + 2):
        _gather_wait(j, j % 4)
    plsc.subcore_barrier()
    pltpu.sync_copy(acc_sh.at[pl.ds(s * ACC_TILE, ACC_TILE)],
                    outp_hbm.at[pl.ds(c * ACC_ROWS + s * ACC_TILE, ACC_TILE)])


_R = 2048
_GRID = NROW // _R

_x_spec = pl.BlockSpec((_R, D), lambda i: (i, 0))
_hp_spec = pl.BlockSpec((2, _R, H), lambda i: (0, i, 0))
_half0_spec = pl.BlockSpec((1, _R, H), lambda i: (0, i, 0))
_half1_spec = pl.BlockSpec((1, _R, H), lambda i: (1, i, 0))
_deg0_spec = pl.BlockSpec((1, _R, Q), lambda i: (0, i, 0))
_deg1_spec = pl.BlockSpec((1, _R, Q), lambda i: (1, i, 0))
_w_spec = pl.BlockSpec((D, D), lambda i: (0, 0))
_b_spec = pl.BlockSpec((1, D), lambda i: (0, 0))


def _valid_rows(nrows):
    rid = lax.broadcasted_iota(jnp.int32, (_R, 1), 0) + pl.program_id(0) * _R
    return rid < nrows


def _ds_of(d0_ref, d1_ref):
    return lax.rsqrt(d0_ref[0][:, :1] + d1_ref[0][:, :1] + 1.0)


def _split_out(out_ref, hp):
    out_ref[0] = hp[:, :H]
    out_ref[1] = hp[:, H:]


def _tc1_body(x_ref, d0_ref, d1_ref, w_ref, out_ref):
    ds = _ds_of(d0_ref, d1_ref)
    h = lax.dot_general(x_ref[...], w_ref[...], (((1,), (1,)), ((), ())),
                        preferred_element_type=jnp.float32)
    _split_out(out_ref, jnp.where(_valid_rows(N), ds * h, 0.0))


_tc1 = pl.pallas_call(
    _tc1_body,
    grid=(_GRID,),
    in_specs=[_x_spec, _deg0_spec, _deg1_spec, _w_spec],
    out_specs=_hp_spec,
    out_shape=jax.ShapeDtypeStruct((2, NROW, H), jnp.float32),
)


def _tc2_body(a0, a1, h0, h1, d0_ref, d1_ref, b_ref, w_ref, out_ref):
    ds = _ds_of(d0_ref, d1_ref)
    z = jnp.concatenate([a0[0] + h0[0], a1[0] + h1[0]], axis=1)
    h = jnp.tanh(ds * z + b_ref[...])
    hp = ds * lax.dot_general(h, w_ref[...], (((1,), (1,)), ((), ())),
                              preferred_element_type=jnp.float32)
    _split_out(out_ref, jnp.where(_valid_rows(N), hp, 0.0))


_tc2 = pl.pallas_call(
    _tc2_body,
    grid=(_GRID,),
    in_specs=[_half0_spec, _half1_spec, _half0_spec, _half1_spec,
              _deg0_spec, _deg1_spec, _b_spec, _w_spec],
    out_specs=_hp_spec,
    out_shape=jax.ShapeDtypeStruct((2, NROW, H), jnp.float32),
)


def _tc3_body(a0, a1, h0, h1, d0_ref, d1_ref, b_ref, w_ref, br_ref, out_ref):
    ds = _ds_of(d0_ref, d1_ref)
    z = jnp.concatenate([a0[0] + h0[0], a1[0] + h1[0]], axis=1)
    h = jnp.tanh(ds * z + b_ref[...])
    y = lax.dot_general(h, w_ref[...], (((1,), (1,)), ((), ())),
                        preferred_element_type=jnp.float32) + br_ref[...]
    out_ref[...] = jax.nn.sigmoid(y) * 0.8 + 0.1


_tc3 = pl.pallas_call(
    _tc3_body,
    grid=(_GRID,),
    in_specs=[_half0_spec, _half1_spec, _half0_spec, _half1_spec,
              _deg0_spec, _deg1_spec, _b_spec, _w_spec, _b_spec],
    out_specs=pl.BlockSpec((_R, D), lambda i: (i, 0)),
    out_shape=jax.ShapeDtypeStruct((N, D), jnp.float32),
)


def kernel(x, edge_index, W1, b1, W2, b2, Wr, br):
    src = edge_index[0].astype(jnp.int32)
    dst = edge_index[1].astype(jnp.int32)

    padd = NW * CD * K - E
    dstd = jnp.concatenate([dst, jnp.full((padd,), DEG_PAD_DST, jnp.int32)])
    dstd = dstd.reshape(NW, CD, K)

    pada = NS * CA * K - E
    srca = jnp.concatenate([src, jnp.full((pada,), AGG_PAD_SRC, jnp.int32)])
    srca = srca.reshape(NS, CA, K)
    dsta = jnp.concatenate([dst, jnp.full((pada,), AGG_PAD_DST, jnp.int32)])
    dsta = dsta.reshape(NS, CA, K)
    extra = jnp.full((NS, 2, K), AGG_PAD_SRC, jnp.int32)
    srca = jnp.concatenate([srca, extra], axis=1)
    dsta = jnp.concatenate([dsta, extra], axis=1)
    srca = jnp.concatenate([srca, srca + NROW], axis=0)

    xp = jnp.concatenate([x, jnp.zeros((NROW - N, D), jnp.float32)])

    onesQ = jnp.ones((K, Q), jnp.float32)
    zerosQ = jnp.zeros((DEG_TILE, Q), jnp.float32)
    zerosH = jnp.zeros((ACC_TILE, H), jnp.float32)
    b1r = b1.reshape(1, D)
    b2r = b2.reshape(1, D)
    brr = br.reshape(1, D)

    degp = _deg_kernel(dstd, onesQ, zerosQ).reshape(NC, DEG_ROWS, Q)
    h1p = _tc1(xp, degp, degp, W1)
    acc1 = _agg_kernel(h1p.reshape(NC * NROW, H), srca, dsta,
                       zerosH).reshape(NC, ACC_ROWS, H)
    h2p = _tc2(acc1, acc1, h1p, h1p, degp, degp, b1r, W2)
    acc2 = _agg_kernel(h2p.reshape(NC * NROW, H), srca, dsta,
                       zerosH).reshape(NC, ACC_ROWS, H)
    return _tc3(acc2, acc2, h2p, h2p, degp, degp, b2r, Wr, brr)
